# baseline (speedup 1.0000x reference)
import functools

import jax
import jax.numpy as jnp
from jax import lax
from jax.experimental import pallas as pl
from jax.experimental.pallas import tpu as pltpu
from jax.experimental.pallas import tpu_sc as plsc

_V = 1000
_VP = 1024
_RB = 8


def _table_body(w_ref, x_ref, y_ref, d_ref, fidx_ref):
    w = w_ref[...]
    m = jnp.max(w, axis=1)
    s = jnp.sum(jnp.exp(w - m[:, None]), axis=1)
    lse = m + jnp.log(s)
    d = lse[:, None] - w
    dp = jnp.concatenate(
        [d, jnp.zeros((_RB, _VP - _V), jnp.float32)], axis=1)
    d_ref[...] = dp.reshape(_RB * _VP)

    @pl.when(pl.program_id(0) == 0)
    def _():
        fidx_ref[...] = x_ref[...] * _VP + y_ref[...]


def _tc_stage(w, x, y):
    nb = w.shape[0] // _RB
    return pl.pallas_call(
        _table_body,
        grid=(nb,),
        in_specs=[
            pl.BlockSpec((_RB, _V), lambda i: (i, 0)),
            pl.BlockSpec(x.shape, lambda i: (0, 0)),
            pl.BlockSpec(x.shape, lambda i: (0, 0)),
        ],
        out_specs=(
            pl.BlockSpec((_RB * _VP,), lambda i: (i,)),
            pl.BlockSpec(x.shape, lambda i: (0, 0)),
        ),
        out_shape=(
            jax.ShapeDtypeStruct((w.shape[0] * _VP,), jnp.float32),
            jax.ShapeDtypeStruct(x.shape, jnp.int32),
        ),
    )(w, x, y)


def _make_sc_gather(tok, nc, ns):
    nw = nc * ns
    per_w = tok // nw
    assert tok % nw == 0 and per_w % 16 == 0
    ch = 80
    nchunk = per_w // ch
    mesh = plsc.VectorSubcoreMesh(core_axis_name="c", subcore_axis_name="s")

    @functools.partial(
        pl.kernel,
        out_type=jax.ShapeDtypeStruct((tok,), jnp.float32),
        mesh=mesh,
        scratch_types=[
            pltpu.VMEM((per_w,), jnp.int32),
            pltpu.VMEM((per_w,), jnp.float32),
            pltpu.SemaphoreType.DMA,
        ],
    )
    def sc_kernel(fidx_hbm, d_hbm, out_hbm, fidx, outv, sem):
        wid = lax.axis_index("s") * nc + lax.axis_index("c")
        base = wid * per_w
        pltpu.sync_copy(fidx_hbm.at[pl.ds(base, per_w)], fidx)
        copies = []
        for c in range(nchunk):
            sl = pl.ds(c * ch, ch)
            copies.append(pltpu.async_copy(d_hbm.at[fidx.at[sl]], outv.at[sl], sem))
        for cp in copies:
            cp.wait()
        pltpu.sync_copy(outv, out_hbm.at[pl.ds(base, per_w)])

    return sc_kernel


def kernel(x, y, w_embed):
    b, t = x.shape
    tok = b * t
    info = plsc.get_sparse_core_info()
    d_flat, fidx = _tc_stage(w_embed, x, y)
    sc = _make_sc_gather(tok, info.num_cores, info.num_subcores)
    loss = sc(fidx.reshape(-1), d_flat)
    return loss.reshape(b, t)


# speedup vs baseline: 2.4192x; 2.4192x over previous
import functools

import jax
import jax.numpy as jnp
from jax import lax
from jax.experimental import pallas as pl
from jax.experimental.pallas import tpu as pltpu
from jax.experimental.pallas import tpu_sc as plsc

_V = 1000


def _table_body(w_ref, x_ref, y_ref, d_ref, fidx_ref):
    w = w_ref[...]
    m = jnp.max(w, axis=1)
    s = jnp.sum(jnp.exp(w - m[:, None]), axis=1)
    lse = m + jnp.log(s)
    d_ref[...] = lse[:, None] - w
    fidx_ref[...] = x_ref[...] * _V + y_ref[...]


def _tc_stage(w, x, y):
    return pl.pallas_call(
        _table_body,
        out_shape=(
            jax.ShapeDtypeStruct(w.shape, jnp.float32),
            jax.ShapeDtypeStruct(x.shape, jnp.int32),
        ),
    )(w, x, y)


def _make_sc_gather(tok, nc, ns):
    nw = nc * ns
    per_w = tok // nw
    assert tok % nw == 0 and per_w % 16 == 0
    ch = 80
    nchunk = per_w // ch
    mesh = plsc.VectorSubcoreMesh(core_axis_name="c", subcore_axis_name="s")

    @functools.partial(
        pl.kernel,
        out_type=jax.ShapeDtypeStruct((tok,), jnp.float32),
        mesh=mesh,
        scratch_types=[
            pltpu.VMEM((per_w,), jnp.int32),
            pltpu.VMEM((per_w,), jnp.float32),
            pltpu.SemaphoreType.DMA,
        ],
    )
    def sc_kernel(fidx_hbm, d_hbm, out_hbm, fidx, outv, sem):
        wid = lax.axis_index("s") * nc + lax.axis_index("c")
        base = wid * per_w
        pltpu.sync_copy(fidx_hbm.at[pl.ds(base, per_w)], fidx)
        copies = []
        for c in range(nchunk):
            sl = pl.ds(c * ch, ch)
            copies.append(pltpu.async_copy(d_hbm.at[fidx.at[sl]], outv.at[sl], sem))
        for cp in copies:
            cp.wait()
        pltpu.sync_copy(outv, out_hbm.at[pl.ds(base, per_w)])

    return sc_kernel


def kernel(x, y, w_embed):
    b, t = x.shape
    tok = b * t
    info = plsc.get_sparse_core_info()
    d_tab, fidx = _tc_stage(w_embed, x, y)
    sc = _make_sc_gather(tok, info.num_cores, info.num_subcores)
    loss = sc(fidx.reshape(-1), d_tab.reshape(-1))
    return loss.reshape(b, t)


# speedup vs baseline: 2.7979x; 1.1565x over previous
import functools

import jax
import jax.numpy as jnp
from jax import lax
from jax.experimental import pallas as pl
from jax.experimental.pallas import tpu as pltpu
from jax.experimental.pallas import tpu_sc as plsc

_V = 1000


def _table_body(w_ref, x_ref, y_ref, d_ref, fidx_ref):
    w = w_ref[...]
    m = jnp.max(w, axis=1)
    s = jnp.sum(jnp.exp(w - m[:, None]), axis=1)
    lse = m + jnp.log(s)
    d_ref[...] = lse[:, None] - w
    fidx_ref[...] = x_ref[...] * _V + y_ref[...]


def _tc_stage(w, x, y):
    return pl.pallas_call(
        _table_body,
        out_shape=(
            jax.ShapeDtypeStruct(w.shape, jnp.float32),
            jax.ShapeDtypeStruct(x.shape, jnp.int32),
        ),
    )(w, x, y)


def _make_sc_gather(tok, nc, ns):
    nw = nc * ns
    per_w = tok // nw
    assert tok % nw == 0 and per_w % 16 == 0
    ch = 80
    nchunk = per_w // ch
    mesh = plsc.VectorSubcoreMesh(core_axis_name="c", subcore_axis_name="s")

    @functools.partial(
        pl.kernel,
        out_type=jax.ShapeDtypeStruct((tok,), jnp.float32),
        mesh=mesh,
        scratch_types=[
            pltpu.VMEM((per_w,), jnp.int32),
            pltpu.VMEM((per_w,), jnp.float32),
            pltpu.SemaphoreType.DMA,
        ],
    )
    def sc_kernel(fidx_hbm, d_hbm, out_hbm, fidx, outv, sem):
        wid = lax.axis_index("s") * nc + lax.axis_index("c")
        base = wid * per_w
        pltpu.sync_copy(fidx_hbm.at[pl.ds(base, per_w)], fidx)
        copies = []
        for c in range(nchunk):
            sl = pl.ds(c * ch, ch)
            copies.append(pltpu.async_copy(d_hbm.at[fidx.at[sl]], outv.at[sl], sem))
        for cp in copies:
            cp.wait()
        pltpu.sync_copy(outv, out_hbm.at[pl.ds(base, per_w)])

    return sc_kernel


def kernel(x, y, w_embed):
    b, t = x.shape
    tok = b * t
    info = plsc.get_sparse_core_info()
    d_tab, fidx = _tc_stage(w_embed, jnp.swapaxes(x, 0, 1),
                            jnp.swapaxes(y, 0, 1))
    sc = _make_sc_gather(tok, info.num_cores, info.num_subcores)
    loss = sc(fidx.reshape(-1), d_tab.reshape(-1))
    return jnp.swapaxes(loss.reshape(t, b), 0, 1)


# speedup vs baseline: 2.8845x; 1.0310x over previous
import functools

import jax
import jax.numpy as jnp
from jax import lax
from jax.experimental import pallas as pl
from jax.experimental.pallas import tpu as pltpu
from jax.experimental.pallas import tpu_sc as plsc

_V = 1000
_L = 128
_CT = 8


def _lse_body(w_ref, x_ref, y_ref, lse_ref, fidx_ref):
    w = w_ref[...]
    m = jnp.max(w, axis=1)
    s = jnp.sum(jnp.exp(w - m[:, None]), axis=1)
    lse = m + jnp.log(s)
    lse_ref[...] = jnp.broadcast_to(lse[:, None], (_V, _L))
    x = x_ref[...]
    y = y_ref[...]
    fidx_ref[...] = (y >> 7) * (_V * _L) + x * _L + (y & (_L - 1))


def _tc_lse(w, x, y):
    return pl.pallas_call(
        _lse_body,
        out_shape=(
            jax.ShapeDtypeStruct((_V, _L), jnp.float32),
            jax.ShapeDtypeStruct(x.shape, jnp.int32),
        ),
    )(w, x, y)


def _dtab_body(w_ref, lse_ref, d_ref):
    d_ref[...] = lse_ref[...] - w_ref[...]


def _tc_dtab(w, lse2d):
    return pl.pallas_call(
        _dtab_body,
        grid=(_CT,),
        in_specs=[
            pl.BlockSpec((_V, _L), lambda ct: (0, ct)),
            pl.BlockSpec((_V, _L), lambda ct: (0, 0)),
        ],
        out_specs=pl.BlockSpec((_V, _L), lambda ct: (ct, 0)),
        out_shape=jax.ShapeDtypeStruct((_CT * _V, _L), jnp.float32),
    )(w, lse2d)


def _make_sc_gather(tok, nc, ns):
    nw = nc * ns
    per_w = tok // nw
    assert tok % nw == 0 and per_w % 16 == 0
    ch = 80
    nchunk = per_w // ch
    mesh = plsc.VectorSubcoreMesh(core_axis_name="c", subcore_axis_name="s")

    @functools.partial(
        pl.kernel,
        out_type=jax.ShapeDtypeStruct((tok,), jnp.float32),
        mesh=mesh,
        scratch_types=[
            pltpu.VMEM((per_w,), jnp.int32),
            pltpu.VMEM((per_w,), jnp.float32),
            pltpu.SemaphoreType.DMA,
        ],
    )
    def sc_kernel(fidx_hbm, d_hbm, out_hbm, fidx, outv, sem):
        wid = lax.axis_index("s") * nc + lax.axis_index("c")
        base = wid * per_w
        pltpu.sync_copy(fidx_hbm.at[pl.ds(base, per_w)], fidx)
        copies = []
        for c in range(nchunk):
            sl = pl.ds(c * ch, ch)
            copies.append(pltpu.async_copy(d_hbm.at[fidx.at[sl]], outv.at[sl], sem))
        for cp in copies:
            cp.wait()
        pltpu.sync_copy(outv, out_hbm.at[pl.ds(base, per_w)])

    return sc_kernel


def kernel(x, y, w_embed):
    b, t = x.shape
    tok = b * t
    info = plsc.get_sparse_core_info()
    lse2d, fidx = _tc_lse(w_embed, jnp.swapaxes(x, 0, 1),
                          jnp.swapaxes(y, 0, 1))
    d_tab = _tc_dtab(w_embed, lse2d)
    sc = _make_sc_gather(tok, info.num_cores, info.num_subcores)
    loss = sc(fidx.reshape(-1), d_tab.reshape(-1))
    return jnp.swapaxes(loss.reshape(t, b), 0, 1)


# speedup vs baseline: 3.4670x; 1.2019x over previous
import functools

import jax
import jax.numpy as jnp
from jax import lax
from jax.experimental import pallas as pl
from jax.experimental.pallas import tpu as pltpu
from jax.experimental.pallas import tpu_sc as plsc

_V = 1000
_L = 128
_CT = 8


def _table_body(w_ref, x_ref, y_ref, d_ref, fidx_ref):
    w = w_ref[...]
    m = jnp.max(w, axis=1)
    s = jnp.sum(jnp.exp(w - m[:, None]), axis=1)
    lse = m + jnp.log(s)
    lse_bc = jnp.broadcast_to(lse[:, None], (_V, _L))
    for ct in range(_CT - 1):
        d_ref[pl.ds(ct * _V, _V), :] = lse_bc - w[:, ct * _L:(ct + 1) * _L]
    wlast = jnp.concatenate(
        [w[:, (_CT - 1) * _L:],
         jnp.zeros((_V, _CT * _L - _V), jnp.float32)], axis=1)
    d_ref[pl.ds((_CT - 1) * _V, _V), :] = lse_bc - wlast
    x = x_ref[...]
    y = y_ref[...]
    fidx_ref[...] = (y >> 7) * (_V * _L) + x * _L + (y & (_L - 1))


def _tc_stage(w, x, y):
    return pl.pallas_call(
        _table_body,
        out_shape=(
            jax.ShapeDtypeStruct((_CT * _V, _L), jnp.float32),
            jax.ShapeDtypeStruct(x.shape, jnp.int32),
        ),
    )(w, x, y)


def _make_sc_gather(tok, nc, ns):
    nw = nc * ns
    per_w = tok // nw
    assert tok % nw == 0 and per_w % 16 == 0
    ch = 80
    nchunk = per_w // ch
    mesh = plsc.VectorSubcoreMesh(core_axis_name="c", subcore_axis_name="s")

    @functools.partial(
        pl.kernel,
        out_type=jax.ShapeDtypeStruct((tok,), jnp.float32),
        mesh=mesh,
        scratch_types=[
            pltpu.VMEM((per_w,), jnp.int32),
            pltpu.VMEM((per_w,), jnp.float32),
            pltpu.SemaphoreType.DMA,
        ],
    )
    def sc_kernel(fidx_hbm, d_hbm, out_hbm, fidx, outv, sem):
        wid = lax.axis_index("s") * nc + lax.axis_index("c")
        base = wid * per_w
        pltpu.sync_copy(fidx_hbm.at[pl.ds(base, per_w)], fidx)
        copies = []
        for c in range(nchunk):
            sl = pl.ds(c * ch, ch)
            copies.append(pltpu.async_copy(d_hbm.at[fidx.at[sl]], outv.at[sl], sem))
        for cp in copies:
            cp.wait()
        pltpu.sync_copy(outv, out_hbm.at[pl.ds(base, per_w)])

    return sc_kernel


def kernel(x, y, w_embed):
    b, t = x.shape
    tok = b * t
    info = plsc.get_sparse_core_info()
    d_tab, fidx = _tc_stage(w_embed, jnp.swapaxes(x, 0, 1),
                            jnp.swapaxes(y, 0, 1))
    sc = _make_sc_gather(tok, info.num_cores, info.num_subcores)
    loss = sc(fidx.reshape(-1), d_tab.reshape(-1))
    return jnp.swapaxes(loss.reshape(t, b), 0, 1)


# speedup vs baseline: 3.6948x; 1.0657x over previous
import functools

import jax
import jax.numpy as jnp
from jax import lax
from jax.experimental import pallas as pl
from jax.experimental.pallas import tpu as pltpu
from jax.experimental.pallas import tpu_sc as plsc

_V = 1000
_L = 128
_CT = 8


def _table_body(w_ref, x_ref, y_ref, d_ref, fidx_ref):
    w = w_ref[...]
    m = jnp.max(w, axis=1)
    s = jnp.sum(jnp.exp(w - m[:, None]), axis=1)
    lse = m + jnp.log(s)
    lse_bc = jnp.broadcast_to(lse[:, None], (_V, _L))
    for ct in range(_CT - 1):
        d_ref[pl.ds(ct * _V, _V), :] = lse_bc - w[:, ct * _L:(ct + 1) * _L]
    wlast = jnp.concatenate(
        [w[:, (_CT - 1) * _L:],
         jnp.zeros((_V, _CT * _L - _V), jnp.float32)], axis=1)
    d_ref[pl.ds((_CT - 1) * _V, _V), :] = lse_bc - wlast
    x = x_ref[...]
    y = y_ref[...]
    fidx = (y >> 7) * (_V * _L) + x * _L + (y & (_L - 1))
    fidx_ref[...] = fidx.reshape(fidx_ref.shape)


def _tc_stage(w, x, y):
    return pl.pallas_call(
        _table_body,
        out_shape=(
            jax.ShapeDtypeStruct((_CT * _V, _L), jnp.float32),
            jax.ShapeDtypeStruct((x.shape[0] * x.shape[1],), jnp.int32),
        ),
    )(w, x, y)


def _make_sc_gather(tok, nc, ns):
    nw = nc * ns
    per_w = tok // nw
    assert tok % nw == 0 and per_w % 16 == 0
    chunks = []
    off = 0
    while off < per_w:
        c = min(128, per_w - off)
        chunks.append((off, c))
        off += c
    mesh = plsc.VectorSubcoreMesh(core_axis_name="c", subcore_axis_name="s")

    @functools.partial(
        pl.kernel,
        out_type=jax.ShapeDtypeStruct((tok,), jnp.float32),
        mesh=mesh,
        scratch_types=[
            pltpu.VMEM((per_w,), jnp.int32),
            pltpu.VMEM((per_w,), jnp.float32),
            pltpu.SemaphoreType.DMA,
        ],
    )
    def sc_kernel(fidx_hbm, d_hbm, out_hbm, fidx, outv, sem):
        wid = lax.axis_index("s") * nc + lax.axis_index("c")
        base = wid * per_w
        pltpu.sync_copy(fidx_hbm.at[pl.ds(base, per_w)], fidx)
        copies = []
        for off, c in chunks:
            sl = pl.ds(off, c)
            copies.append(pltpu.async_copy(d_hbm.at[fidx.at[sl]], outv.at[sl], sem))
        for cp in copies:
            cp.wait()
        pltpu.sync_copy(outv, out_hbm.at[pl.ds(base, per_w)])

    return sc_kernel


def kernel(x, y, w_embed):
    b, t = x.shape
    tok = b * t
    info = plsc.get_sparse_core_info()
    d_tab, fidx = _tc_stage(w_embed, jnp.swapaxes(x, 0, 1),
                            jnp.swapaxes(y, 0, 1))
    sc = _make_sc_gather(tok, info.num_cores, info.num_subcores)
    loss = sc(fidx, d_tab.reshape(-1))
    return jnp.swapaxes(loss.reshape(t, b), 0, 1)
